# baseline (speedup 1.0000x reference)
import jax
import jax.numpy as jnp
from jax import lax
from jax.experimental import pallas as pl
from jax.experimental.pallas import tpu as pltpu

BM = 1024
BN = 1024
BK = 512


def _linear_kernel(x_ref, w_ref, b_ref, o_ref):
    k = pl.program_id(2)
    xb = x_ref[...].astype(jnp.bfloat16)
    wb = w_ref[...].astype(jnp.bfloat16)
    acc = lax.dot_general(
        xb, wb, (((1,), (1,)), ((), ())), preferred_element_type=jnp.float32
    )

    @pl.when(k == 0)
    def _init():
        o_ref[...] = acc + b_ref[...]

    @pl.when(k != 0)
    def _accum():
        o_ref[...] += acc


def kernel(input, weight, bias):
    m, kdim = input.shape
    n, _ = weight.shape
    bias2 = bias.reshape(1, n)
    grid = (m // BM, n // BN, kdim // BK)
    return pl.pallas_call(
        _linear_kernel,
        grid=grid,
        in_specs=[
            pl.BlockSpec((BM, BK), lambda i, j, k: (i, k)),
            pl.BlockSpec((BN, BK), lambda i, j, k: (j, k)),
            pl.BlockSpec((1, BN), lambda i, j, k: (0, j)),
        ],
        out_specs=pl.BlockSpec((BM, BN), lambda i, j, k: (i, j)),
        out_shape=jax.ShapeDtypeStruct((m, n), jnp.float32),
        compiler_params=pltpu.CompilerParams(
            dimension_semantics=("parallel", "parallel", "arbitrary"),
        ),
    )(input, weight, bias2)


# speedup vs baseline: 1.0056x; 1.0056x over previous
import jax
import jax.numpy as jnp
from jax import lax
from jax.experimental import pallas as pl
from jax.experimental.pallas import tpu as pltpu

BM = 2048
BN = 2048
BK = 256


def _linear_kernel(x_ref, w_ref, b_ref, o_ref):
    k = pl.program_id(2)
    xb = x_ref[...].astype(jnp.bfloat16)
    wb = w_ref[...].astype(jnp.bfloat16)
    acc = lax.dot_general(
        xb, wb, (((1,), (1,)), ((), ())), preferred_element_type=jnp.float32
    )

    @pl.when(k == 0)
    def _init():
        o_ref[...] = acc + b_ref[...]

    @pl.when(k != 0)
    def _accum():
        o_ref[...] += acc


def kernel(input, weight, bias):
    m, kdim = input.shape
    n, _ = weight.shape
    bias2 = bias.reshape(1, n)
    grid = (m // BM, n // BN, kdim // BK)
    return pl.pallas_call(
        _linear_kernel,
        grid=grid,
        in_specs=[
            pl.BlockSpec((BM, BK), lambda i, j, k: (i, k)),
            pl.BlockSpec((BN, BK), lambda i, j, k: (j, k)),
            pl.BlockSpec((1, BN), lambda i, j, k: (0, j)),
        ],
        out_specs=pl.BlockSpec((BM, BN), lambda i, j, k: (i, j)),
        out_shape=jax.ShapeDtypeStruct((m, n), jnp.float32),
        compiler_params=pltpu.CompilerParams(
            dimension_semantics=("parallel", "parallel", "arbitrary"),
        ),
    )(input, weight, bias2)


# speedup vs baseline: 1.6193x; 1.6102x over previous
import jax
import jax.numpy as jnp
from jax import lax
from jax.experimental import pallas as pl
from jax.experimental.pallas import tpu as pltpu

BM = 2048
BN = 2048
BK = 512


def _linear_kernel(x_ref, w_ref, b_ref, o_ref):
    k = pl.program_id(2)

    @pl.when(k == 0)
    def _init():
        o_ref[...] = jnp.broadcast_to(b_ref[...], o_ref.shape)

    xb = x_ref[...].astype(jnp.bfloat16)
    wb = w_ref[...].astype(jnp.bfloat16)
    o_ref[...] += lax.dot_general(
        xb, wb, (((1,), (1,)), ((), ())), preferred_element_type=jnp.float32
    )


def kernel(input, weight, bias):
    m, kdim = input.shape
    n, _ = weight.shape
    bias2 = bias.reshape(1, n)
    grid = (m // BM, n // BN, kdim // BK)
    return pl.pallas_call(
        _linear_kernel,
        grid=grid,
        in_specs=[
            pl.BlockSpec((BM, BK), lambda i, j, k: (i, k)),
            pl.BlockSpec((BN, BK), lambda i, j, k: (j, k)),
            pl.BlockSpec((1, BN), lambda i, j, k: (0, j)),
        ],
        out_specs=pl.BlockSpec((BM, BN), lambda i, j, k: (i, j)),
        out_shape=jax.ShapeDtypeStruct((m, n), jnp.float32),
        compiler_params=pltpu.CompilerParams(
            dimension_semantics=("parallel", "parallel", "arbitrary"),
        ),
    )(input, weight, bias2)


# speedup vs baseline: 1.6393x; 1.0123x over previous
import jax
import jax.numpy as jnp
from jax import lax
from jax.experimental import pallas as pl
from jax.experimental.pallas import tpu as pltpu

BM = 2048
BN = 2048
BK = 512


def _linear_kernel(x_ref, w_ref, b_ref, o_ref):
    k = pl.program_id(2)
    xb = x_ref[...].astype(jnp.bfloat16)
    wb = w_ref[...].astype(jnp.bfloat16)
    base = jnp.where(
        k == 0, jnp.broadcast_to(b_ref[...], o_ref.shape), o_ref[...]
    )
    o_ref[...] = base + lax.dot_general(
        xb, wb, (((1,), (1,)), ((), ())), preferred_element_type=jnp.float32
    )


def kernel(input, weight, bias):
    m, kdim = input.shape
    n, _ = weight.shape
    bias2 = bias.reshape(1, n)
    grid = (m // BM, n // BN, kdim // BK)
    return pl.pallas_call(
        _linear_kernel,
        grid=grid,
        in_specs=[
            pl.BlockSpec((BM, BK), lambda i, j, k: (i, k)),
            pl.BlockSpec((BN, BK), lambda i, j, k: (j, k)),
            pl.BlockSpec((1, BN), lambda i, j, k: (0, j)),
        ],
        out_specs=pl.BlockSpec((BM, BN), lambda i, j, k: (i, j)),
        out_shape=jax.ShapeDtypeStruct((m, n), jnp.float32),
        compiler_params=pltpu.CompilerParams(
            dimension_semantics=("parallel", "parallel", "arbitrary"),
        ),
    )(input, weight, bias2)


# speedup vs baseline: 1.6472x; 1.0049x over previous
import jax
import jax.numpy as jnp
from jax import lax
from jax.experimental import pallas as pl
from jax.experimental.pallas import tpu as pltpu

FILL = 16
BM = 256


def _linear_kernel(x_ref, w_ref, b_ref, o_ref, ws_ref):
    t = pl.program_id(0)
    rs = w_ref.shape[0]

    @pl.when(t < FILL)
    def _fill():
        ws_ref[pl.ds(t * rs, rs), :] = w_ref[...].astype(jnp.bfloat16)

    @pl.when(t >= FILL)
    def _compute():
        xb = x_ref[...].astype(jnp.bfloat16)
        o_ref[...] = lax.dot_general(
            xb, ws_ref[...], (((1,), (1,)), ((), ())),
            preferred_element_type=jnp.float32,
        ) + b_ref[...]


def kernel(input, weight, bias):
    m, kdim = input.shape
    n, _ = weight.shape
    bias2 = bias.reshape(1, n)
    nsteps = FILL + m // BM
    return pl.pallas_call(
        _linear_kernel,
        grid=(nsteps,),
        in_specs=[
            pl.BlockSpec((BM, kdim), lambda t: (jnp.maximum(t - FILL, 0), 0)),
            pl.BlockSpec((n // FILL, kdim), lambda t: (jnp.minimum(t, FILL - 1), 0)),
            pl.BlockSpec((1, n), lambda t: (0, 0)),
        ],
        out_specs=pl.BlockSpec((BM, n), lambda t: (jnp.maximum(t - FILL, 0), 0)),
        out_shape=jax.ShapeDtypeStruct((m, n), jnp.float32),
        scratch_shapes=[pltpu.VMEM((n, kdim), jnp.bfloat16)],
        compiler_params=pltpu.CompilerParams(
            dimension_semantics=("arbitrary",),
        ),
    )(input, weight, bias2)
